# baseline (speedup 1.0000x reference)
import functools

import jax
import jax.numpy as jnp
from jax import lax
from jax.experimental import pallas as pl
from jax.experimental.pallas import tpu as pltpu
from jax.experimental.pallas import tpu_sc as plsc

ROWS, COLS = 8192, 4096
K = 16
FEAT, PATCH = 64, 16
NC, NS, L = 2, 16, 16
NW = NC * NS
ROWS_SC = 4608
RPW = ROWS_SC // NW
CHUNK = 8
NBUF = 3
TBLK = 512
ROWS_TC = ROWS - ROWS_SC

_mesh = plsc.VectorSubcoreMesh(
    core_axis_name="c", subcore_axis_name="s", num_cores=NC, num_subcores=NS)


@functools.partial(
    pl.kernel,
    out_type=[
        jax.ShapeDtypeStruct((NW, COLS), jnp.float32),
        jax.ShapeDtypeStruct((NW * L,), jnp.int32),
    ],
    mesh=_mesh,
    compiler_params=pltpu.CompilerParams(needs_layout_passes=False),
    scratch_types=[
        pltpu.VMEM((RPW,), jnp.int32),
        pltpu.VMEM((RPW,), jnp.int32),
        pltpu.VMEM((COLS,), jnp.float32),
        pltpu.VMEM((NBUF, CHUNK, COLS), jnp.float32),
        pltpu.SemaphoreType.DMA((NBUF,)),
    ],
)
def _sc_gather_sum(sim, mask, part, cnt,
                   mask_v, lidx_v, acc_v, rows_v, sems):
    w = lax.axis_index("s") * NC + lax.axis_index("c")
    base = w * RPW
    pltpu.sync_copy(mask.at[pl.ds(base, RPW)], mask_v)

    zf = jnp.zeros((L,), jnp.float32)
    zi = jnp.zeros((L,), jnp.int32)

    def _zero_acc(i, carry):
        acc_v[pl.ds(i * L, L)] = zf
        return carry
    lax.fori_loop(0, COLS // L, _zero_acc, 0, unroll=8)

    def _zero_idx(i, carry):
        lidx_v[pl.ds(i * L, L)] = zi
        return carry
    lax.fori_loop(0, RPW // L, _zero_idx, 0, unroll=4)

    iota = lax.iota(jnp.int32, L)

    def _compact(j, off):
        mi = mask_v[pl.ds(j * L, L)]
        mb = mi != 0
        csum = plsc.cumsum(mi)
        pos = csum - mi + off
        ids = iota + (base + j * L)
        plsc.store_scatter(lidx_v, [pos], ids, mask=mb)
        return off + jnp.max(csum)

    cnt_w = lax.fori_loop(0, RPW // L, _compact, jnp.int32(0))

    nch = (cnt_w + (CHUNK - 1)) // CHUNK

    def _issue(g, slot):
        idx_ref = lidx_v.at[pl.ds(g * CHUNK, CHUNK)]
        pltpu.async_copy(sim.at[idx_ref], rows_v.at[slot], sems.at[slot])

    def _wait(g, slot):
        idx_ref = lidx_v.at[pl.ds(g * CHUNK, CHUNK)]
        pltpu.make_async_copy(sim.at[idx_ref], rows_v.at[slot],
                              sems.at[slot]).wait()

    for b in range(NBUF):
        @pl.when(b < nch)
        def _():
            _issue(b, b)

    def _outer(o, carry):
        for b in range(NBUF):
            g = o * NBUF + b

            @pl.when(g < nch)
            def _():
                _wait(g, b)
                for r in range(CHUNK):
                    @pl.when(g * CHUNK + r >= cnt_w)
                    def _():
                        def _zrow(q, c3):
                            rows_v[b, r, pl.ds(q * L, L)] = zf
                            return c3
                        lax.fori_loop(0, COLS // L, _zrow, 0, unroll=8)

                @plsc.parallel_loop(0, COLS // L, unroll=4)
                def _acc_chunk(q):
                    o2 = q * L
                    t0 = rows_v[b, 0, pl.ds(o2, L)] + rows_v[b, 1, pl.ds(o2, L)]
                    t1 = rows_v[b, 2, pl.ds(o2, L)] + rows_v[b, 3, pl.ds(o2, L)]
                    t2 = rows_v[b, 4, pl.ds(o2, L)] + rows_v[b, 5, pl.ds(o2, L)]
                    t3 = rows_v[b, 6, pl.ds(o2, L)] + rows_v[b, 7, pl.ds(o2, L)]
                    plsc.addupdate(acc_v.at[pl.ds(o2, L)],
                                   (t0 + t1) + (t2 + t3))

                @pl.when(g + NBUF < nch)
                def _():
                    _issue(g + NBUF, b)
        return carry

    lax.fori_loop(0, (nch + (NBUF - 1)) // NBUF, _outer, 0)

    pltpu.sync_copy(acc_v, part.at[w])
    mask_v[pl.ds(0, L)] = zi + cnt_w
    pltpu.sync_copy(mask_v.at[pl.ds(0, L)], cnt.at[pl.ds(w * L, L)])


def _tc_dense_kernel(mask_ref, sim_ref, sum_ref, cnt_ref, acc_ref, c_ref):
    i = pl.program_id(0)

    @pl.when(i == 0)
    def _init():
        acc_ref[...] = jnp.zeros_like(acc_ref)
        c_ref[0, 0] = jnp.float32(0.0)

    m = mask_ref[...]
    blk = sim_ref[...]
    acc_ref[...] += jnp.dot(m, blk, preferred_element_type=jnp.float32)
    c_ref[0, 0] += jnp.sum(m)

    @pl.when(i == pl.num_programs(0) - 1)
    def _final():
        sum_ref[...] = acc_ref[...]
        cnt_ref[0, 0] = c_ref[0, 0]


def _tc_finish_kernel(part_ref, cnt_ref, tcp_ref, tcc_ref,
                      avg_ref, idx_ref, scores_ref, points_ref, sq_ref):
    t = tcp_ref[...] + jnp.sum(part_ref[...], axis=0, keepdims=True)
    csc = (jnp.sum(cnt_ref[...]).astype(jnp.float32) / L)
    cnt = csc + tcc_ref[0, 0]
    avg = t / cnt
    avg_ref[...] = avg

    NR = 32
    CW = COLS // NR
    for r in range(NR):
        sq_ref[r:r + 1, :] = avg[:, r * CW:(r + 1) * CW]
    a = sq_ref[...]
    col = (jax.lax.broadcasted_iota(jnp.int32, (NR, CW), 0) * CW
           + jax.lax.broadcasted_iota(jnp.int32, (NR, CW), 1))
    idxs = []
    scs = []
    for _ in range(K):
        mn = jnp.min(a)
        sel = jnp.where(a == mn, col, jnp.int32(COLS))
        ix = jnp.min(sel)
        a = jnp.where(col == ix, jnp.float32(jnp.inf), a)
        idxs.append(ix)
        scs.append(mn)
    idxv = jnp.stack(idxs)
    scv = jnp.stack(scs)
    idx_ref[0, :] = idxv
    scores_ref[0, :] = scv
    xf = (idxv % FEAT).astype(jnp.float32) * PATCH + (PATCH // 2)
    yf = (idxv // FEAT).astype(jnp.float32) * PATCH + (PATCH // 2)
    points_ref[0, :] = xf
    points_ref[1, :] = yf
    points_ref[2, :] = scv


@jax.jit
def kernel(similarity_map, ref_mask):
    mask_i32 = ref_mask.astype(jnp.int32)
    mask_f = ref_mask.astype(jnp.float32).reshape(1, ROWS)

    part_sc, cnt_sc = _sc_gather_sum(similarity_map, mask_i32)

    tc_sum, tc_cnt = pl.pallas_call(
        _tc_dense_kernel,
        grid=(ROWS_TC // TBLK,),
        in_specs=[
            pl.BlockSpec((1, TBLK), lambda i: (0, i + ROWS_SC // TBLK)),
            pl.BlockSpec((TBLK, COLS), lambda i: (i + ROWS_SC // TBLK, 0)),
        ],
        out_specs=[
            pl.BlockSpec((1, COLS), lambda i: (0, 0)),
            pl.BlockSpec((1, 1), lambda i: (0, 0), memory_space=pltpu.SMEM),
        ],
        out_shape=[
            jax.ShapeDtypeStruct((1, COLS), jnp.float32),
            jax.ShapeDtypeStruct((1, 1), jnp.float32),
        ],
        scratch_shapes=[
            pltpu.VMEM((1, COLS), jnp.float32),
            pltpu.SMEM((1, 1), jnp.float32),
        ],
    )(mask_f, similarity_map)

    avg2, idx, scores, points = pl.pallas_call(
        _tc_finish_kernel,
        out_shape=[
            jax.ShapeDtypeStruct((1, COLS), jnp.float32),
            jax.ShapeDtypeStruct((1, K), jnp.int32),
            jax.ShapeDtypeStruct((1, K), jnp.float32),
            jax.ShapeDtypeStruct((3, K), jnp.float32),
        ],
        scratch_shapes=[pltpu.VMEM((32, COLS // 32), jnp.float32)],
    )(part_sc, cnt_sc.reshape(1, NW * L), tc_sum, tc_cnt)

    return (avg2.reshape(COLS), idx.reshape(K), scores.reshape(K), points.T)


# speedup vs baseline: 1.0254x; 1.0254x over previous
import functools

import jax
import jax.numpy as jnp
from jax import lax
from jax.experimental import pallas as pl
from jax.experimental.pallas import tpu as pltpu
from jax.experimental.pallas import tpu_sc as plsc

ROWS, COLS = 8192, 4096
K = 16
FEAT, PATCH = 64, 16
NC, NS, L = 2, 16, 16
NW = NC * NS
ROWS_SC = 5632
RPW = ROWS_SC // NW
CHUNK = 8
NBUF = 3
TBLK = 512
ROWS_TC = ROWS - ROWS_SC

_mesh = plsc.VectorSubcoreMesh(
    core_axis_name="c", subcore_axis_name="s", num_cores=NC, num_subcores=NS)


@functools.partial(
    pl.kernel,
    out_type=[
        jax.ShapeDtypeStruct((NW, COLS), jnp.float32),
        jax.ShapeDtypeStruct((NW * L,), jnp.int32),
    ],
    mesh=_mesh,
    compiler_params=pltpu.CompilerParams(needs_layout_passes=False),
    scratch_types=[
        pltpu.VMEM((RPW,), jnp.int32),
        pltpu.VMEM((RPW,), jnp.int32),
        pltpu.VMEM((COLS,), jnp.float32),
        pltpu.VMEM((NBUF, CHUNK, COLS), jnp.float32),
        pltpu.SemaphoreType.DMA((NBUF,)),
    ],
)
def _sc_gather_sum(sim, mask, part, cnt,
                   mask_v, lidx_v, acc_v, rows_v, sems):
    w = lax.axis_index("s") * NC + lax.axis_index("c")
    base = w * RPW
    pltpu.sync_copy(mask.at[pl.ds(base, RPW)], mask_v)

    zf = jnp.zeros((L,), jnp.float32)
    zi = jnp.zeros((L,), jnp.int32)

    def _zero_acc(i, carry):
        acc_v[pl.ds(i * L, L)] = zf
        return carry
    lax.fori_loop(0, COLS // L, _zero_acc, 0, unroll=8)

    def _zero_idx(i, carry):
        lidx_v[pl.ds(i * L, L)] = zi
        return carry
    lax.fori_loop(0, RPW // L, _zero_idx, 0, unroll=4)

    iota = lax.iota(jnp.int32, L)

    def _compact(j, off):
        mi = mask_v[pl.ds(j * L, L)]
        mb = mi != 0
        csum = plsc.cumsum(mi)
        pos = csum - mi + off
        ids = iota + (base + j * L)
        plsc.store_scatter(lidx_v, [pos], ids, mask=mb)
        return off + jnp.max(csum)

    cnt_w = lax.fori_loop(0, RPW // L, _compact, jnp.int32(0))

    nch = (cnt_w + (CHUNK - 1)) // CHUNK

    def _issue(g, slot):
        idx_ref = lidx_v.at[pl.ds(g * CHUNK, CHUNK)]
        pltpu.async_copy(sim.at[idx_ref], rows_v.at[slot], sems.at[slot])

    def _wait(g, slot):
        idx_ref = lidx_v.at[pl.ds(g * CHUNK, CHUNK)]
        pltpu.make_async_copy(sim.at[idx_ref], rows_v.at[slot],
                              sems.at[slot]).wait()

    for b in range(NBUF):
        @pl.when(b < nch)
        def _():
            _issue(b, b)

    def _outer(o, carry):
        for b in range(NBUF):
            g = o * NBUF + b

            @pl.when(g < nch)
            def _():
                _wait(g, b)
                for r in range(CHUNK):
                    @pl.when(g * CHUNK + r >= cnt_w)
                    def _():
                        def _zrow(q, c3):
                            rows_v[b, r, pl.ds(q * L, L)] = zf
                            return c3
                        lax.fori_loop(0, COLS // L, _zrow, 0, unroll=8)

                @plsc.parallel_loop(0, COLS // L, unroll=4)
                def _acc_chunk(q):
                    o2 = q * L
                    t0 = rows_v[b, 0, pl.ds(o2, L)] + rows_v[b, 1, pl.ds(o2, L)]
                    t1 = rows_v[b, 2, pl.ds(o2, L)] + rows_v[b, 3, pl.ds(o2, L)]
                    t2 = rows_v[b, 4, pl.ds(o2, L)] + rows_v[b, 5, pl.ds(o2, L)]
                    t3 = rows_v[b, 6, pl.ds(o2, L)] + rows_v[b, 7, pl.ds(o2, L)]
                    plsc.addupdate(acc_v.at[pl.ds(o2, L)],
                                   (t0 + t1) + (t2 + t3))

                @pl.when(g + NBUF < nch)
                def _():
                    _issue(g + NBUF, b)
        return carry

    lax.fori_loop(0, (nch + (NBUF - 1)) // NBUF, _outer, 0)

    pltpu.sync_copy(acc_v, part.at[w])
    mask_v[pl.ds(0, L)] = zi + cnt_w
    pltpu.sync_copy(mask_v.at[pl.ds(0, L)], cnt.at[pl.ds(w * L, L)])


def _tc_dense_kernel(mask_ref, sim_ref, sum_ref, cnt_ref, acc_ref, c_ref):
    i = pl.program_id(0)

    @pl.when(i == 0)
    def _init():
        acc_ref[...] = jnp.zeros_like(acc_ref)
        c_ref[0, 0] = jnp.float32(0.0)

    m = mask_ref[...]
    blk = sim_ref[...]
    acc_ref[...] += jnp.dot(m, blk, preferred_element_type=jnp.float32)
    c_ref[0, 0] += jnp.sum(m)

    @pl.when(i == pl.num_programs(0) - 1)
    def _final():
        sum_ref[...] = acc_ref[...]
        cnt_ref[0, 0] = c_ref[0, 0]


def _tc_finish_kernel(part_ref, cnt_ref, tcp_ref, tcc_ref,
                      avg_ref, idx_ref, scores_ref, points_ref, sq_ref):
    t = tcp_ref[...] + jnp.sum(part_ref[...], axis=0, keepdims=True)
    csc = (jnp.sum(cnt_ref[...]).astype(jnp.float32) / L)
    cnt = csc + tcc_ref[0, 0]
    avg = t / cnt
    avg_ref[...] = avg

    NR = 32
    CW = COLS // NR
    for r in range(NR):
        sq_ref[r:r + 1, :] = avg[:, r * CW:(r + 1) * CW]
    a = sq_ref[...]
    col = (jax.lax.broadcasted_iota(jnp.int32, (NR, CW), 0) * CW
           + jax.lax.broadcasted_iota(jnp.int32, (NR, CW), 1))
    idxs = []
    scs = []
    for _ in range(K):
        mn = jnp.min(a)
        sel = jnp.where(a == mn, col, jnp.int32(COLS))
        ix = jnp.min(sel)
        a = jnp.where(col == ix, jnp.float32(jnp.inf), a)
        idxs.append(ix)
        scs.append(mn)
    idxv = jnp.stack(idxs)
    scv = jnp.stack(scs)
    idx_ref[0, :] = idxv
    scores_ref[0, :] = scv
    xf = (idxv % FEAT).astype(jnp.float32) * PATCH + (PATCH // 2)
    yf = (idxv // FEAT).astype(jnp.float32) * PATCH + (PATCH // 2)
    points_ref[0, :] = xf
    points_ref[1, :] = yf
    points_ref[2, :] = scv


@jax.jit
def kernel(similarity_map, ref_mask):
    mask_i32 = ref_mask.astype(jnp.int32)
    mask_f = ref_mask.astype(jnp.float32).reshape(1, ROWS)

    part_sc, cnt_sc = _sc_gather_sum(similarity_map, mask_i32)

    tc_sum, tc_cnt = pl.pallas_call(
        _tc_dense_kernel,
        grid=(ROWS_TC // TBLK,),
        in_specs=[
            pl.BlockSpec((1, TBLK), lambda i: (0, i + ROWS_SC // TBLK)),
            pl.BlockSpec((TBLK, COLS), lambda i: (i + ROWS_SC // TBLK, 0)),
        ],
        out_specs=[
            pl.BlockSpec((1, COLS), lambda i: (0, 0)),
            pl.BlockSpec((1, 1), lambda i: (0, 0), memory_space=pltpu.SMEM),
        ],
        out_shape=[
            jax.ShapeDtypeStruct((1, COLS), jnp.float32),
            jax.ShapeDtypeStruct((1, 1), jnp.float32),
        ],
        scratch_shapes=[
            pltpu.VMEM((1, COLS), jnp.float32),
            pltpu.SMEM((1, 1), jnp.float32),
        ],
    )(mask_f, similarity_map)

    avg2, idx, scores, points = pl.pallas_call(
        _tc_finish_kernel,
        out_shape=[
            jax.ShapeDtypeStruct((1, COLS), jnp.float32),
            jax.ShapeDtypeStruct((1, K), jnp.int32),
            jax.ShapeDtypeStruct((1, K), jnp.float32),
            jax.ShapeDtypeStruct((3, K), jnp.float32),
        ],
        scratch_shapes=[pltpu.VMEM((32, COLS // 32), jnp.float32)],
    )(part_sc, cnt_sc.reshape(1, NW * L), tc_sum, tc_cnt)

    return (avg2.reshape(COLS), idx.reshape(K), scores.reshape(K), points.T)


# speedup vs baseline: 1.0695x; 1.0430x over previous
import functools

import jax
import jax.numpy as jnp
from jax import lax
from jax.experimental import pallas as pl
from jax.experimental.pallas import tpu as pltpu
from jax.experimental.pallas import tpu_sc as plsc

ROWS, COLS = 8192, 4096
K = 16
FEAT, PATCH = 64, 16
NC, NS, L = 2, 16, 16
NW = NC * NS
ROWS_SC = 5120
RPW = ROWS_SC // NW
CHUNK = 8
NBUF = 3
TBLK = 512
ROWS_TC = ROWS - ROWS_SC

_mesh = plsc.VectorSubcoreMesh(
    core_axis_name="c", subcore_axis_name="s", num_cores=NC, num_subcores=NS)


@functools.partial(
    pl.kernel,
    out_type=[
        jax.ShapeDtypeStruct((NW, COLS), jnp.float32),
        jax.ShapeDtypeStruct((NW * L,), jnp.int32),
    ],
    mesh=_mesh,
    compiler_params=pltpu.CompilerParams(needs_layout_passes=False),
    scratch_types=[
        pltpu.VMEM((RPW,), jnp.int32),
        pltpu.VMEM((RPW,), jnp.int32),
        pltpu.VMEM((COLS,), jnp.float32),
        pltpu.VMEM((NBUF, CHUNK, COLS), jnp.float32),
        pltpu.SemaphoreType.DMA((NBUF,)),
    ],
)
def _sc_gather_sum(sim, mask, part, cnt,
                   mask_v, lidx_v, acc_v, rows_v, sems):
    w = lax.axis_index("s") * NC + lax.axis_index("c")
    base = w * RPW
    pltpu.sync_copy(mask.at[pl.ds(base, RPW)], mask_v)

    zf = jnp.zeros((L,), jnp.float32)
    zi = jnp.zeros((L,), jnp.int32)

    def _zero_acc(i, carry):
        acc_v[pl.ds(i * L, L)] = zf
        return carry
    lax.fori_loop(0, COLS // L, _zero_acc, 0, unroll=8)

    def _zero_idx(i, carry):
        lidx_v[pl.ds(i * L, L)] = zi
        return carry
    lax.fori_loop(0, RPW // L, _zero_idx, 0, unroll=4)

    iota = lax.iota(jnp.int32, L)

    def _compact(j, off):
        mi = mask_v[pl.ds(j * L, L)]
        mb = mi != 0
        csum = plsc.cumsum(mi)
        pos = csum - mi + off
        ids = iota + (base + j * L)
        plsc.store_scatter(lidx_v, [pos], ids, mask=mb)
        return off + jnp.max(csum)

    cnt_w = lax.fori_loop(0, RPW // L, _compact, jnp.int32(0))

    nch = (cnt_w + (CHUNK - 1)) // CHUNK

    def _issue(g, slot):
        idx_ref = lidx_v.at[pl.ds(g * CHUNK, CHUNK)]
        pltpu.async_copy(sim.at[idx_ref], rows_v.at[slot], sems.at[slot])

    def _wait(g, slot):
        idx_ref = lidx_v.at[pl.ds(g * CHUNK, CHUNK)]
        pltpu.make_async_copy(sim.at[idx_ref], rows_v.at[slot],
                              sems.at[slot]).wait()

    for b in range(NBUF):
        @pl.when(b < nch)
        def _():
            _issue(b, b)

    def _outer(o, carry):
        for b in range(NBUF):
            g = o * NBUF + b

            @pl.when(g < nch)
            def _():
                _wait(g, b)
                for r in range(CHUNK):
                    @pl.when(g * CHUNK + r >= cnt_w)
                    def _():
                        def _zrow(q, c3):
                            rows_v[b, r, pl.ds(q * L, L)] = zf
                            return c3
                        lax.fori_loop(0, COLS // L, _zrow, 0, unroll=8)

                @plsc.parallel_loop(0, COLS // L, unroll=4)
                def _acc_chunk(q):
                    o2 = q * L
                    t0 = rows_v[b, 0, pl.ds(o2, L)] + rows_v[b, 1, pl.ds(o2, L)]
                    t1 = rows_v[b, 2, pl.ds(o2, L)] + rows_v[b, 3, pl.ds(o2, L)]
                    t2 = rows_v[b, 4, pl.ds(o2, L)] + rows_v[b, 5, pl.ds(o2, L)]
                    t3 = rows_v[b, 6, pl.ds(o2, L)] + rows_v[b, 7, pl.ds(o2, L)]
                    plsc.addupdate(acc_v.at[pl.ds(o2, L)],
                                   (t0 + t1) + (t2 + t3))

                @pl.when(g + NBUF < nch)
                def _():
                    _issue(g + NBUF, b)
        return carry

    lax.fori_loop(0, (nch + (NBUF - 1)) // NBUF, _outer, 0)

    pltpu.sync_copy(acc_v, part.at[w])
    mask_v[pl.ds(0, L)] = zi + cnt_w
    pltpu.sync_copy(mask_v.at[pl.ds(0, L)], cnt.at[pl.ds(w * L, L)])


def _tc_dense_kernel(mask_ref, sim_ref, sum_ref, cnt_ref, acc_ref, c_ref):
    i = pl.program_id(0)

    @pl.when(i == 0)
    def _init():
        acc_ref[...] = jnp.zeros_like(acc_ref)
        c_ref[0, 0] = jnp.float32(0.0)

    m = mask_ref[...]
    blk = sim_ref[...]
    acc_ref[...] += jnp.dot(m, blk, preferred_element_type=jnp.float32)
    c_ref[0, 0] += jnp.sum(m)

    @pl.when(i == pl.num_programs(0) - 1)
    def _final():
        sum_ref[...] = acc_ref[...]
        cnt_ref[0, 0] = c_ref[0, 0]


def _tc_finish_kernel(part_ref, cnt_ref, tcp_ref, tcc_ref,
                      avg_ref, idx_ref, scores_ref, points_ref, sq_ref):
    t = tcp_ref[...] + jnp.sum(part_ref[...], axis=0, keepdims=True)
    csc = (jnp.sum(cnt_ref[...]).astype(jnp.float32) / L)
    cnt = csc + tcc_ref[0, 0]
    avg = t / cnt
    avg_ref[...] = avg

    NR = 32
    CW = COLS // NR
    for r in range(NR):
        sq_ref[r:r + 1, :] = avg[:, r * CW:(r + 1) * CW]
    a = sq_ref[...]
    col = (jax.lax.broadcasted_iota(jnp.int32, (NR, CW), 0) * CW
           + jax.lax.broadcasted_iota(jnp.int32, (NR, CW), 1))
    idxs = []
    scs = []
    for _ in range(K):
        mn = jnp.min(a)
        sel = jnp.where(a == mn, col, jnp.int32(COLS))
        ix = jnp.min(sel)
        a = jnp.where(col == ix, jnp.float32(jnp.inf), a)
        idxs.append(ix)
        scs.append(mn)
    idxv = jnp.stack(idxs)
    scv = jnp.stack(scs)
    idx_ref[0, :] = idxv
    scores_ref[0, :] = scv
    xf = (idxv % FEAT).astype(jnp.float32) * PATCH + (PATCH // 2)
    yf = (idxv // FEAT).astype(jnp.float32) * PATCH + (PATCH // 2)
    points_ref[0, :] = xf
    points_ref[1, :] = yf
    points_ref[2, :] = scv


@jax.jit
def kernel(similarity_map, ref_mask):
    mask_i32 = ref_mask.astype(jnp.int32)
    mask_f = ref_mask.astype(jnp.float32).reshape(1, ROWS)

    part_sc, cnt_sc = _sc_gather_sum(similarity_map, mask_i32)

    tc_sum, tc_cnt = pl.pallas_call(
        _tc_dense_kernel,
        grid=(ROWS_TC // TBLK,),
        in_specs=[
            pl.BlockSpec((1, TBLK), lambda i: (0, i + ROWS_SC // TBLK)),
            pl.BlockSpec((TBLK, COLS), lambda i: (i + ROWS_SC // TBLK, 0)),
        ],
        out_specs=[
            pl.BlockSpec((1, COLS), lambda i: (0, 0)),
            pl.BlockSpec((1, 1), lambda i: (0, 0), memory_space=pltpu.SMEM),
        ],
        out_shape=[
            jax.ShapeDtypeStruct((1, COLS), jnp.float32),
            jax.ShapeDtypeStruct((1, 1), jnp.float32),
        ],
        scratch_shapes=[
            pltpu.VMEM((1, COLS), jnp.float32),
            pltpu.SMEM((1, 1), jnp.float32),
        ],
    )(mask_f, similarity_map)

    avg2, idx, scores, points = pl.pallas_call(
        _tc_finish_kernel,
        out_shape=[
            jax.ShapeDtypeStruct((1, COLS), jnp.float32),
            jax.ShapeDtypeStruct((1, K), jnp.int32),
            jax.ShapeDtypeStruct((1, K), jnp.float32),
            jax.ShapeDtypeStruct((3, K), jnp.float32),
        ],
        scratch_shapes=[pltpu.VMEM((32, COLS // 32), jnp.float32)],
    )(part_sc, cnt_sc.reshape(1, NW * L), tc_sum, tc_cnt)

    return (avg2.reshape(COLS), idx.reshape(K), scores.reshape(K), points.T)
